# Initial kernel scaffold; baseline (speedup 1.0000x reference)
import jax
import jax.numpy as jnp
from jax.experimental import pallas as pl


def kernel(user_emb, item_emb, group_emb, uh_rows, uh_cols, uh_vals, ih_rows, ih_cols, ih_vals, fh_rows, fh_cols, fh_vals, W0, b0, W1, b1, num_users, num_items):
    raise NotImplementedError("write your pallas kernel here")


# speedup vs baseline: 1.8283x; 1.8283x over previous
import functools

import jax
import jax.numpy as jnp
from jax import lax
from jax.experimental import pallas as pl
from jax.experimental.pallas import tpu as pltpu
from jax.experimental.pallas import tpu_sc as plsc

U = 60000
I = 40000
G = 10000
D = 128

NC = 2
NS = 16
NW = NC * NS

EB = 128
WR_G = 320
GP = WR_G * NW
WR_E = 640
WPW_E = 5
EP = WR_E * WPW_E * NW
NWIN_E = WPW_E * NW


def _mesh():
    return plsc.VectorSubcoreMesh(
        core_axis_name="c", subcore_axis_name="s", num_cores=NC, num_subcores=NS
    )


def _sload(ref, j):
    return ref[pl.ds(j, 16)][0]


def _zero_acc(acc, wr):
    z = jnp.zeros((16,), jnp.float32)

    def row(r, carry):
        for c in range(8):
            acc[r, pl.ds(c * 16, 16)] = z
        return carry

    lax.fori_loop(0, wr, row, 0)


def _process_window(row_base, est, eend, rows_hbm, cols_hbm, vals_hbm, table_hbm,
                    tab_off, acc, colv, rowv, valv, gbuf, sem, wr):
    abase = (est // 8) * 8
    nb = (eend - abase + EB - 1) // EB

    def batch(bi, carry):
        base = abase + bi * EB
        pltpu.sync_copy(cols_hbm.at[pl.ds(base, EB)], colv.at[pl.ds(0, EB)])
        pltpu.sync_copy(rows_hbm.at[pl.ds(base, EB)], rowv.at[pl.ds(0, EB)])
        pltpu.sync_copy(vals_hbm.at[pl.ds(base, EB)], valv.at[pl.ds(0, EB)])
        if tab_off:
            for c in range(EB // 16):
                colv[pl.ds(c * 16, 16)] = colv[pl.ds(c * 16, 16)] + tab_off
        pltpu.async_copy(table_hbm.at[colv.at[pl.ds(0, EB)]], gbuf, sem).wait()

        def edge(j, carry2):
            e = base + j
            ok = (e >= est) & (e < eend)
            val = jnp.where(ok, _sload(valv, j), 0.0)
            lr = jnp.minimum(jnp.maximum(_sload(rowv, j) - row_base, 0), wr - 1)
            for c in range(8):
                g = gbuf[j, pl.ds(c * 16, 16)]
                plsc.addupdate(acc.at[lr, pl.ds(c * 16, 16)], g * val)
            return carry2

        lax.fori_loop(0, EB, edge, 0)
        return carry

    lax.fori_loop(0, nb, batch, 0)


def _stage_a(itab_off, uh_rows, uh_cols, uh_vals, ih_rows, ih_cols, ih_vals,
             utab, itab, ubounds, ibounds):
    def body(ur, uc, uv, ir_, ic, iv, utab_h, itab_h, ub_h, ib_h,
             umsg, imsg, bu_v, bi_v, colv, rowv, valv, gbuf, acc, sem):
        wid = lax.axis_index("s") * NC + lax.axis_index("c")
        pltpu.sync_copy(ub_h, bu_v)
        pltpu.sync_copy(ib_h, bi_v)
        row_base = wid * WR_G

        _zero_acc(acc, WR_G)
        _process_window(row_base, _sload(bu_v, wid), _sload(bu_v, wid + 1),
                        ur, uc, uv, utab_h,
                        0, acc, colv, rowv, valv, gbuf, sem, WR_G)
        pltpu.sync_copy(acc, umsg.at[pl.ds(row_base, WR_G)])

        _zero_acc(acc, WR_G)
        _process_window(row_base, _sload(bi_v, wid), _sload(bi_v, wid + 1),
                        ir_, ic, iv, itab_h,
                        itab_off, acc, colv, rowv, valv, gbuf, sem, WR_G)
        pltpu.sync_copy(acc, imsg.at[pl.ds(row_base, WR_G)])

    f = pl.kernel(
        body,
        out_type=(
            jax.ShapeDtypeStruct((GP, D), jnp.float32),
            jax.ShapeDtypeStruct((GP, D), jnp.float32),
        ),
        mesh=_mesh(),
        scratch_types=(
            pltpu.VMEM((48,), jnp.int32),
            pltpu.VMEM((48,), jnp.int32),
            pltpu.VMEM((EB + 16,), jnp.int32),
            pltpu.VMEM((EB + 16,), jnp.int32),
            pltpu.VMEM((EB + 16,), jnp.float32),
            pltpu.VMEM((EB, D), jnp.float32),
            pltpu.VMEM((WR_G, D), jnp.float32),
            pltpu.SemaphoreType.DMA,
        ),
    )
    return f(uh_rows, uh_cols, uh_vals, ih_rows, ih_cols, ih_vals,
             utab, itab, ubounds, ibounds)


def _stage_c(emit_emb, fh_rows, fh_cols, fh_vals, msg_tab, fbounds, accin):
    def body(fr, fc, fv, msg_h, fb_h, accin_h, *rest):
        if emit_emb:
            emb_o, acc_o, bf_v, colv, rowv, valv, gbuf, acc, idxv, sem = rest
        else:
            acc_o, bf_v, colv, rowv, valv, gbuf, acc, idxv, sem = rest
        wid = lax.axis_index("s") * NC + lax.axis_index("c")
        pltpu.sync_copy(fb_h, bf_v)
        for win in range(WPW_E):
            gwin = wid * WPW_E + win
            row_base = gwin * WR_E
            est = _sload(bf_v, gwin)
            eend = _sload(bf_v, gwin + 1)
            if emit_emb:
                _zero_acc(acc, WR_E)
            else:
                pltpu.sync_copy(accin_h.at[pl.ds(row_base, WR_E)], acc)
            _process_window(row_base, est, eend, fr, fc, fv, msg_h,
                            0, acc, colv, rowv, valv, gbuf, sem, WR_E)
            if emit_emb:
                pltpu.sync_copy(acc, emb_o.at[pl.ds(row_base, WR_E)])
                def fill(k, carry):
                    idxv[pl.ds(k * 16, 16)] = (
                        row_base + k * 16 + lax.iota(jnp.int32, 16))
                    return carry
                lax.fori_loop(0, WR_E // 16, fill, 0)
                for k in range(WR_E // EB):
                    pltpu.async_copy(
                        accin_h.at[idxv.at[pl.ds(k * EB, EB)]],
                        acc.at[pl.ds(k * EB, EB)], sem, add=True).wait()
                pltpu.sync_copy(acc, acc_o.at[pl.ds(row_base, WR_E)])
            else:
                pltpu.sync_copy(acc, acc_o.at[pl.ds(row_base, WR_E)])

    outs = [jax.ShapeDtypeStruct((EP, D), jnp.float32)]
    if emit_emb:
        outs = [jax.ShapeDtypeStruct((EP, D), jnp.float32)] + outs
    f = pl.kernel(
        body,
        out_type=tuple(outs),
        mesh=_mesh(),
        scratch_types=(
            pltpu.VMEM((184,), jnp.int32),
            pltpu.VMEM((EB + 16,), jnp.int32),
            pltpu.VMEM((EB + 16,), jnp.int32),
            pltpu.VMEM((EB + 16,), jnp.float32),
            pltpu.VMEM((EB, D), jnp.float32),
            pltpu.VMEM((WR_E, D), jnp.float32),
            pltpu.VMEM((WR_E,), jnp.int32),
            pltpu.SemaphoreType.DMA,
        ),
    )
    return f(fh_rows, fh_cols, fh_vals, msg_tab, fbounds, accin)


RB = 1024


def _mm_body(u_ref, i_ref, g_ref, hein_ref, wu_ref, wi_ref, wg_ref, b_ref,
             msg_ref, he_ref):
    u = u_ref[...]
    it = i_ref[...]
    ge = g_ref[...]
    m = jnp.dot(u, wu_ref[...], preferred_element_type=jnp.float32)
    m = m + jnp.dot(it, wi_ref[...], preferred_element_type=jnp.float32)
    m = m + jnp.dot(it * ge, wg_ref[...], preferred_element_type=jnp.float32)
    m = m + b_ref[...]
    msg_ref[...] = m
    he_ref[...] = hein_ref[...] + m


def _stage_b(umsg, imsg, gep, hein, W, b):
    wu, wi, wg = W[:D], W[D:2 * D], W[2 * D:]
    b2 = b.reshape(1, D)
    row_spec = pl.BlockSpec((RB, D), lambda ib: (ib, 0))
    w_spec = pl.BlockSpec((D, D), lambda ib: (0, 0))
    return pl.pallas_call(
        _mm_body,
        grid=(GP // RB,),
        in_specs=[row_spec, row_spec, row_spec, row_spec, w_spec, w_spec,
                  w_spec, pl.BlockSpec((1, D), lambda ib: (0, 0))],
        out_specs=[row_spec, row_spec],
        out_shape=(
            jax.ShapeDtypeStruct((GP, D), jnp.float32),
            jax.ShapeDtypeStruct((GP, D), jnp.float32),
        ),
    )(umsg, imsg, gep, hein, wu, wi, wg, b2)


def kernel(user_emb, item_emb, group_emb, uh_rows, uh_cols, uh_vals,
           ih_rows, ih_cols, ih_vals, fh_rows, fh_cols, fh_vals,
           W0, b0, W1, b1, num_users, num_items):
    i32 = jnp.int32

    def padded(x, n=EB, cval=0):
        return jnp.pad(x, (0, n), constant_values=cval)

    uhr, uhc, uhv = padded(uh_rows), padded(uh_cols), padded(uh_vals)
    ihr, ihc, ihv = padded(ih_rows), padded(ih_cols), padded(ih_vals)
    fhr, fhc, fhv = padded(fh_rows), padded(fh_cols), padded(fh_vals)

    ub = jnp.pad(jnp.searchsorted(uh_rows, jnp.arange(NW + 1, dtype=i32) * WR_G)
                 .astype(i32), (0, 48 - (NW + 1)))
    ib = jnp.pad(jnp.searchsorted(ih_rows, jnp.arange(NW + 1, dtype=i32) * WR_G)
                 .astype(i32), (0, 48 - (NW + 1)))
    fb = jnp.pad(jnp.searchsorted(fh_rows, jnp.arange(NWIN_E + 1, dtype=i32) * WR_E)
                 .astype(i32), (0, 184 - (NWIN_E + 1)))

    base_p = jnp.pad(jnp.concatenate([user_emb, item_emb], axis=0),
                     ((0, EP - (U + I)), (0, 0)))
    gep = jnp.pad(group_emb, ((0, GP - G), (0, 0)))

    um1, im1 = _stage_a(0, uhr, uhc, uhv, ihr, ihc, ihv,
                        user_emb, item_emb, ub, ib)
    msg1, he1 = _stage_b(um1, im1, gep, gep, W0, b0)
    emb1, r1 = _stage_c(True, fhr, fhc, fhv, msg1, fb, base_p)

    um2, im2 = _stage_a(U, uhr, uhc, uhv, ihr, ihc, ihv, emb1, emb1, ub, ib)
    msg2, he2 = _stage_b(um2, im2, gep, he1, W1, b1)
    (final_p,) = _stage_c(False, fhr, fhc, fhv, msg2, fb, r1)

    return final_p[:U + I], he2[:G]


# speedup vs baseline: 2.2842x; 1.2494x over previous
import jax
import jax.numpy as jnp
from jax import lax
from jax.experimental import pallas as pl
from jax.experimental.pallas import tpu as pltpu
from jax.experimental.pallas import tpu_sc as plsc

U = 60000
I = 40000
G = 10000
D = 128

NC = 2
NS = 16
NW = NC * NS

EB = 128
EPAD = 4 * EB
WR_G = 320
GP = WR_G * NW
WR_E = 640
WPW_E = 5
EP = WR_E * WPW_E * NW
NWIN_E = WPW_E * NW


def _mesh():
    return plsc.VectorSubcoreMesh(
        core_axis_name="c", subcore_axis_name="s", num_cores=NC, num_subcores=NS
    )


def _sload(ref, j):
    return ref[pl.ds(j, 16)][0]


def _zero_acc(acc, wr):
    z = jnp.zeros((16,), jnp.float32)

    def row(r, carry):
        for c in range(8):
            acc[r, pl.ds(c * 16, 16)] = z
        return carry

    lax.fori_loop(0, wr, row, 0)


class _Buf:
    def __init__(self, colv, rowv, valv, gbuf, sem):
        self.colv, self.rowv, self.valv, self.gbuf, self.sem = (
            colv, rowv, valv, gbuf, sem)


def _issue(bi, buf, ctx):
    (abase, estv, eendv, rbv, wr, tab_off, rows_hbm, cols_hbm, vals_hbm,
     table_hbm, iotas) = ctx
    base = abase + bi * EB
    pltpu.sync_copy(cols_hbm.at[pl.ds(base, EB)], buf.colv)
    pltpu.sync_copy(rows_hbm.at[pl.ds(base, EB)], buf.rowv)
    pltpu.sync_copy(vals_hbm.at[pl.ds(base, EB)], buf.valv)
    basev = jnp.full((16,), 0, jnp.int32) + base
    zero = jnp.zeros((16,), jnp.float32)
    for c in range(8):
        ds = pl.ds(c * 16, 16)
        if tab_off:
            buf.colv[ds] = buf.colv[ds] + tab_off
        ev = basev + iotas[c]
        m = (ev >= estv) & (ev < eendv)
        buf.valv[ds] = jnp.where(m, buf.valv[ds], zero)
        lr = buf.rowv[ds] - rbv
        buf.rowv[ds] = jnp.minimum(jnp.maximum(lr, 0), wr - 1)
    pltpu.async_copy(table_hbm.at[buf.colv], buf.gbuf, buf.sem)


def _wait(buf, ctx):
    table_hbm = ctx[9]
    pltpu.make_async_copy(table_hbm.at[buf.colv], buf.gbuf, buf.sem).wait()


def _edges(buf, acc, iotas):
    def grp(j4, carry):
        for k in range(4):
            j = j4 * 4 + k
            jv = jnp.full((16,), 0, jnp.int32) + j
            valsp = plsc.load_gather(buf.valv, [jv])
            lrsp = plsc.load_gather(buf.rowv, [jv])
            for c in range(8):
                g = buf.gbuf[j, pl.ds(c * 16, 16)]
                plsc.addupdate_scatter(acc, [lrsp, iotas[c]], g * valsp)
        return carry

    lax.fori_loop(0, EB // 4, grp, 0)


def _process_window(row_base, est, eend, rows_hbm, cols_hbm, vals_hbm,
                    table_hbm, tab_off, acc, b0, b1, iotas, wr):
    abase = (est // 8) * 8
    nb = (eend - abase + EB - 1) // EB
    nb2 = (nb + 1) // 2
    estv = jnp.full((16,), 0, jnp.int32) + est
    eendv = jnp.full((16,), 0, jnp.int32) + eend
    rbv = jnp.full((16,), 0, jnp.int32) + row_base
    ctx = (abase, estv, eendv, rbv, wr, tab_off, rows_hbm, cols_hbm,
           vals_hbm, table_hbm, iotas)

    _issue(0, b0, ctx)

    def pair(i2, carry):
        b = 2 * i2
        _issue(b + 1, b1, ctx)
        _wait(b0, ctx)
        _edges(b0, acc, iotas)
        _issue(b + 2, b0, ctx)
        _wait(b1, ctx)
        _edges(b1, acc, iotas)
        return carry

    lax.fori_loop(0, nb2, pair, 0)
    _wait(b0, ctx)


def _sc_scratch(wr):
    return (
        pltpu.VMEM((EB,), jnp.int32),
        pltpu.VMEM((EB,), jnp.int32),
        pltpu.VMEM((EB,), jnp.float32),
        pltpu.VMEM((EB, D), jnp.float32),
        pltpu.SemaphoreType.DMA,
        pltpu.VMEM((EB,), jnp.int32),
        pltpu.VMEM((EB,), jnp.int32),
        pltpu.VMEM((EB,), jnp.float32),
        pltpu.VMEM((EB, D), jnp.float32),
        pltpu.SemaphoreType.DMA,
        pltpu.VMEM((wr, D), jnp.float32),
    )


def _iotas():
    return [lax.iota(jnp.int32, 16) + c * 16 for c in range(8)]


def _stage_a(itab_off, uh_rows, uh_cols, uh_vals, ih_rows, ih_cols, ih_vals,
             utab, itab, ubounds, ibounds):
    def body(ur, uc, uv, ir_, ic, iv, utab_h, itab_h, ub_h, ib_h,
             umsg, imsg, *s):
        b0, b1 = _Buf(*s[0:5]), _Buf(*s[5:10])
        acc = s[10]
        bu_v, bi_v = s[11], s[12]
        wid = lax.axis_index("s") * NC + lax.axis_index("c")
        pltpu.sync_copy(ub_h, bu_v)
        pltpu.sync_copy(ib_h, bi_v)
        iotas = _iotas()
        row_base = wid * WR_G

        _zero_acc(acc, WR_G)
        _process_window(row_base, _sload(bu_v, wid), _sload(bu_v, wid + 1),
                        ur, uc, uv, utab_h, 0, acc, b0, b1, iotas, WR_G)
        pltpu.sync_copy(acc, umsg.at[pl.ds(row_base, WR_G)])

        _zero_acc(acc, WR_G)
        _process_window(row_base, _sload(bi_v, wid), _sload(bi_v, wid + 1),
                        ir_, ic, iv, itab_h, itab_off, acc, b0, b1, iotas, WR_G)
        pltpu.sync_copy(acc, imsg.at[pl.ds(row_base, WR_G)])

    f = pl.kernel(
        body,
        out_type=(
            jax.ShapeDtypeStruct((GP, D), jnp.float32),
            jax.ShapeDtypeStruct((GP, D), jnp.float32),
        ),
        mesh=_mesh(),
        compiler_params=pltpu.CompilerParams(needs_layout_passes=False),
        scratch_types=_sc_scratch(WR_G) + (
            pltpu.VMEM((48,), jnp.int32),
            pltpu.VMEM((48,), jnp.int32),
        ),
    )
    return f(uh_rows, uh_cols, uh_vals, ih_rows, ih_cols, ih_vals,
             utab, itab, ubounds, ibounds)


def _stage_c(emit_emb, fh_rows, fh_cols, fh_vals, msg_tab, fbounds, accin):
    def body(fr, fc, fv, msg_h, fb_h, accin_h, *rest):
        if emit_emb:
            emb_o, acc_o = rest[0], rest[1]
            s = rest[2:]
        else:
            acc_o = rest[0]
            s = rest[1:]
        b0, b1 = _Buf(*s[0:5]), _Buf(*s[5:10])
        acc = s[10]
        bf_v, idxv = s[11], s[12]
        wid = lax.axis_index("s") * NC + lax.axis_index("c")
        pltpu.sync_copy(fb_h, bf_v)
        iotas = _iotas()
        for win in range(WPW_E):
            gwin = wid * WPW_E + win
            row_base = gwin * WR_E
            est = _sload(bf_v, gwin)
            eend = _sload(bf_v, gwin + 1)
            if emit_emb:
                _zero_acc(acc, WR_E)
            else:
                pltpu.sync_copy(accin_h.at[pl.ds(row_base, WR_E)], acc)
            _process_window(row_base, est, eend, fr, fc, fv, msg_h,
                            0, acc, b0, b1, iotas, WR_E)
            if emit_emb:
                pltpu.sync_copy(acc, emb_o.at[pl.ds(row_base, WR_E)])
                def fill(k, carry):
                    idxv[pl.ds(k * 16, 16)] = (
                        row_base + k * 16 + lax.iota(jnp.int32, 16))
                    return carry
                lax.fori_loop(0, WR_E // 16, fill, 0)
                for k in range(WR_E // EB):
                    pltpu.async_copy(
                        accin_h.at[idxv.at[pl.ds(k * EB, EB)]],
                        acc.at[pl.ds(k * EB, EB)], b0.sem, add=True).wait()
                pltpu.sync_copy(acc, acc_o.at[pl.ds(row_base, WR_E)])
            else:
                pltpu.sync_copy(acc, acc_o.at[pl.ds(row_base, WR_E)])

    outs = [jax.ShapeDtypeStruct((EP, D), jnp.float32)]
    if emit_emb:
        outs = [jax.ShapeDtypeStruct((EP, D), jnp.float32)] + outs
    f = pl.kernel(
        body,
        out_type=tuple(outs),
        mesh=_mesh(),
        compiler_params=pltpu.CompilerParams(needs_layout_passes=False),
        scratch_types=_sc_scratch(WR_E) + (
            pltpu.VMEM((184,), jnp.int32),
            pltpu.VMEM((WR_E,), jnp.int32),
        ),
    )
    return f(fh_rows, fh_cols, fh_vals, msg_tab, fbounds, accin)


RB = 1024


def _mm_body(u_ref, i_ref, g_ref, hein_ref, wu_ref, wi_ref, wg_ref, b_ref,
             msg_ref, he_ref):
    u = u_ref[...]
    it = i_ref[...]
    ge = g_ref[...]
    m = jnp.dot(u, wu_ref[...], preferred_element_type=jnp.float32)
    m = m + jnp.dot(it, wi_ref[...], preferred_element_type=jnp.float32)
    m = m + jnp.dot(it * ge, wg_ref[...], preferred_element_type=jnp.float32)
    m = m + b_ref[...]
    msg_ref[...] = m
    he_ref[...] = hein_ref[...] + m


def _stage_b(umsg, imsg, gep, hein, W, b):
    wu, wi, wg = W[:D], W[D:2 * D], W[2 * D:]
    b2 = b.reshape(1, D)
    row_spec = pl.BlockSpec((RB, D), lambda ib: (ib, 0))
    w_spec = pl.BlockSpec((D, D), lambda ib: (0, 0))
    return pl.pallas_call(
        _mm_body,
        grid=(GP // RB,),
        in_specs=[row_spec, row_spec, row_spec, row_spec, w_spec, w_spec,
                  w_spec, pl.BlockSpec((1, D), lambda ib: (0, 0))],
        out_specs=[row_spec, row_spec],
        out_shape=(
            jax.ShapeDtypeStruct((GP, D), jnp.float32),
            jax.ShapeDtypeStruct((GP, D), jnp.float32),
        ),
    )(umsg, imsg, gep, hein, wu, wi, wg, b2)


def kernel(user_emb, item_emb, group_emb, uh_rows, uh_cols, uh_vals,
           ih_rows, ih_cols, ih_vals, fh_rows, fh_cols, fh_vals,
           W0, b0, W1, b1, num_users, num_items):
    i32 = jnp.int32

    def padded(x):
        return jnp.pad(x, (0, EPAD))

    uhr, uhc, uhv = padded(uh_rows), padded(uh_cols), padded(uh_vals)
    ihr, ihc, ihv = padded(ih_rows), padded(ih_cols), padded(ih_vals)
    fhr, fhc, fhv = padded(fh_rows), padded(fh_cols), padded(fh_vals)

    ub = jnp.pad(jnp.searchsorted(uh_rows, jnp.arange(NW + 1, dtype=i32) * WR_G)
                 .astype(i32), (0, 48 - (NW + 1)))
    ib = jnp.pad(jnp.searchsorted(ih_rows, jnp.arange(NW + 1, dtype=i32) * WR_G)
                 .astype(i32), (0, 48 - (NW + 1)))
    fb = jnp.pad(jnp.searchsorted(fh_rows, jnp.arange(NWIN_E + 1, dtype=i32) * WR_E)
                 .astype(i32), (0, 184 - (NWIN_E + 1)))

    base_p = jnp.pad(jnp.concatenate([user_emb, item_emb], axis=0),
                     ((0, EP - (U + I)), (0, 0)))
    gep = jnp.pad(group_emb, ((0, GP - G), (0, 0)))

    um1, im1 = _stage_a(0, uhr, uhc, uhv, ihr, ihc, ihv,
                        user_emb, item_emb, ub, ib)
    msg1, he1 = _stage_b(um1, im1, gep, gep, W0, b0)
    emb1, r1 = _stage_c(True, fhr, fhc, fhv, msg1, fb, base_p)

    um2, im2 = _stage_a(U, uhr, uhc, uhv, ihr, ihc, ihv, emb1, emb1, ub, ib)
    msg2, he2 = _stage_b(um2, im2, gep, he1, W1, b1)
    (final_p,) = _stage_c(False, fhr, fhc, fhv, msg2, fb, r1)

    return final_p[:U + I], he2[:G]


# speedup vs baseline: 5.8528x; 2.5623x over previous
import jax
import jax.numpy as jnp
from jax import lax
from jax.experimental import pallas as pl
from jax.experimental.pallas import tpu as pltpu
from jax.experimental.pallas import tpu_sc as plsc

U = 60000
I = 40000
G = 10000
D = 128

NC = 2
NS = 16
NW = NC * NS

EB = 128
SB = 1024
EPAD = SB + EB
WR_G = 320
GP = WR_G * NW
WR_E = 640
WPW_E = 5
EP = WR_E * WPW_E * NW
NWIN_E = WPW_E * NW


def _mesh():
    return plsc.VectorSubcoreMesh(
        core_axis_name="c", subcore_axis_name="s", num_cores=NC, num_subcores=NS
    )


def _sload(ref, j):
    return ref[pl.ds(j, 16)][0]


def _zero_acc(acc, wr):
    z = jnp.zeros((16,), jnp.float32)

    @plsc.parallel_loop(0, wr, step=1, unroll=8)
    def _(r):
        for c in range(8):
            acc[r, pl.ds(c * 16, 16)] = z


class _Buf:
    def __init__(self, colv, rowv, valv, g0, s0, g1, s1):
        self.colv, self.rowv, self.valv = colv, rowv, valv
        self.gbuf = (g0, g1)
        self.sem = (s0, s1)


def _edges(buf, k, p, acc, iotas):
    gbuf = buf.gbuf[p]
    koff = k * EB

    @plsc.parallel_loop(0, EB, step=1, unroll=4)
    def _(j):
        jv = jnp.full((16,), 0, jnp.int32) + (j + koff)
        valsp = plsc.load_gather(buf.valv, [jv])
        lrsp = plsc.load_gather(buf.rowv, [jv])
        for c in range(8):
            g = gbuf[j, pl.ds(c * 16, 16)]
            plsc.addupdate_scatter(acc, [lrsp, iotas[c]], g * valsp)


def _process_window(row_base, est, eend, rows_hbm, cols_hbm, vals_hbm,
                    table_hbm, tab_off, acc, buf, iotas, wr):
    abase = (est // 8) * 8
    nsb = (eend - abase + SB - 1) // SB
    estv = jnp.full((16,), 0, jnp.int32) + est
    eendv = jnp.full((16,), 0, jnp.int32) + eend
    rbv = jnp.full((16,), 0, jnp.int32) + row_base
    iota0 = iotas[0]

    def issue(k, p):
        pltpu.async_copy(
            table_hbm.at[buf.colv.at[pl.ds(k * EB, EB)]],
            buf.gbuf[p], buf.sem[p])

    def wait(k, p):
        pltpu.make_async_copy(
            table_hbm.at[buf.colv.at[pl.ds(k * EB, EB)]],
            buf.gbuf[p], buf.sem[p]).wait()

    def sbatch(si, carry):
        sbase = abase + si * SB
        pltpu.sync_copy(cols_hbm.at[pl.ds(sbase, SB)], buf.colv)
        pltpu.sync_copy(rows_hbm.at[pl.ds(sbase, SB)], buf.rowv)
        pltpu.sync_copy(vals_hbm.at[pl.ds(sbase, SB)], buf.valv)
        zero = jnp.zeros((16,), jnp.float32)

        @plsc.parallel_loop(0, SB // 16, step=1, unroll=4)
        def _(c):
            ds = pl.ds(c * 16, 16)
            if tab_off:
                buf.colv[ds] = buf.colv[ds] + tab_off
            ev = jnp.full((16,), 0, jnp.int32) + (sbase + c * 16) + iota0
            m = (ev >= estv) & (ev < eendv)
            buf.valv[ds] = jnp.where(m, buf.valv[ds], zero)
            lr = buf.rowv[ds] - rbv
            buf.rowv[ds] = jnp.minimum(jnp.maximum(lr, 0), wr - 1)

        nk = jnp.minimum((eend - sbase + EB - 1) // EB, SB // EB)
        nk2 = (nk + 1) // 2

        issue(0, 0)

        def pair(p2, c2):
            k0 = 2 * p2
            pl.when(k0 + 1 < nk)(lambda: issue(k0 + 1, 1))
            wait(k0, 0)
            _edges(buf, k0, 0, acc, iotas)

            @pl.when(k0 + 1 < nk)
            def _():
                pl.when(k0 + 2 < nk)(lambda: issue(k0 + 2, 0))
                wait(k0 + 1, 1)
                _edges(buf, k0 + 1, 1, acc, iotas)
            return c2

        lax.fori_loop(0, nk2, pair, 0)
        return carry

    lax.fori_loop(0, nsb, sbatch, 0)


def _sc_scratch(wr):
    return (
        pltpu.VMEM((SB,), jnp.int32),
        pltpu.VMEM((SB,), jnp.int32),
        pltpu.VMEM((SB,), jnp.float32),
        pltpu.VMEM((EB, D), jnp.float32),
        pltpu.SemaphoreType.DMA,
        pltpu.VMEM((EB, D), jnp.float32),
        pltpu.SemaphoreType.DMA,
        pltpu.VMEM((wr, D), jnp.float32),
    )


def _iotas():
    return [lax.iota(jnp.int32, 16) + c * 16 for c in range(8)]


def _stage_a(itab_off, uh_rows, uh_cols, uh_vals, ih_rows, ih_cols, ih_vals,
             utab, itab, ubounds, ibounds):
    def body(ur, uc, uv, ir_, ic, iv, utab_h, itab_h, ub_h, ib_h,
             umsg, imsg, *s):
        buf = _Buf(*s[0:7])
        acc = s[7]
        bu_v, bi_v = s[8], s[9]
        wid = lax.axis_index("s") * NC + lax.axis_index("c")
        pltpu.sync_copy(ub_h, bu_v)
        pltpu.sync_copy(ib_h, bi_v)
        iotas = _iotas()
        row_base = wid * WR_G

        _zero_acc(acc, WR_G)
        _process_window(row_base, _sload(bu_v, wid), _sload(bu_v, wid + 1),
                        ur, uc, uv, utab_h, 0, acc, buf, iotas, WR_G)
        pltpu.sync_copy(acc, umsg.at[pl.ds(row_base, WR_G)])

        _zero_acc(acc, WR_G)
        _process_window(row_base, _sload(bi_v, wid), _sload(bi_v, wid + 1),
                        ir_, ic, iv, itab_h, itab_off, acc, buf, iotas, WR_G)
        pltpu.sync_copy(acc, imsg.at[pl.ds(row_base, WR_G)])

    f = pl.kernel(
        body,
        out_type=(
            jax.ShapeDtypeStruct((GP, D), jnp.float32),
            jax.ShapeDtypeStruct((GP, D), jnp.float32),
        ),
        mesh=_mesh(),
        compiler_params=pltpu.CompilerParams(needs_layout_passes=False),
        scratch_types=_sc_scratch(WR_G) + (
            pltpu.VMEM((48,), jnp.int32),
            pltpu.VMEM((48,), jnp.int32),
        ),
    )
    return f(uh_rows, uh_cols, uh_vals, ih_rows, ih_cols, ih_vals,
             utab, itab, ubounds, ibounds)


def _stage_c(emit_emb, fh_rows, fh_cols, fh_vals, msg_tab, fbounds, accin):
    def body(fr, fc, fv, msg_h, fb_h, accin_h, *rest):
        if emit_emb:
            emb_o, acc_o = rest[0], rest[1]
            s = rest[2:]
        else:
            acc_o = rest[0]
            s = rest[1:]
        buf = _Buf(*s[0:7])
        acc = s[7]
        bf_v, idxv = s[8], s[9]
        wid = lax.axis_index("s") * NC + lax.axis_index("c")
        pltpu.sync_copy(fb_h, bf_v)
        iotas = _iotas()
        for win in range(WPW_E):
            gwin = wid * WPW_E + win
            row_base = gwin * WR_E
            est = _sload(bf_v, gwin)
            eend = _sload(bf_v, gwin + 1)
            if emit_emb:
                _zero_acc(acc, WR_E)
            else:
                pltpu.sync_copy(accin_h.at[pl.ds(row_base, WR_E)], acc)
            _process_window(row_base, est, eend, fr, fc, fv, msg_h,
                            0, acc, buf, iotas, WR_E)
            if emit_emb:
                pltpu.sync_copy(acc, emb_o.at[pl.ds(row_base, WR_E)])
                def fill(k, carry):
                    idxv[pl.ds(k * 16, 16)] = (
                        row_base + k * 16 + lax.iota(jnp.int32, 16))
                    return carry
                lax.fori_loop(0, WR_E // 16, fill, 0)
                for k in range(WR_E // EB):
                    pltpu.async_copy(
                        accin_h.at[idxv.at[pl.ds(k * EB, EB)]],
                        acc.at[pl.ds(k * EB, EB)], buf.sem[0], add=True).wait()
                pltpu.sync_copy(acc, acc_o.at[pl.ds(row_base, WR_E)])
            else:
                pltpu.sync_copy(acc, acc_o.at[pl.ds(row_base, WR_E)])

    outs = [jax.ShapeDtypeStruct((EP, D), jnp.float32)]
    if emit_emb:
        outs = [jax.ShapeDtypeStruct((EP, D), jnp.float32)] + outs
    f = pl.kernel(
        body,
        out_type=tuple(outs),
        mesh=_mesh(),
        compiler_params=pltpu.CompilerParams(needs_layout_passes=False),
        scratch_types=_sc_scratch(WR_E) + (
            pltpu.VMEM((184,), jnp.int32),
            pltpu.VMEM((WR_E,), jnp.int32),
        ),
    )
    return f(fh_rows, fh_cols, fh_vals, msg_tab, fbounds, accin)


RB = 1024


def _mm_body(u_ref, i_ref, g_ref, hein_ref, wu_ref, wi_ref, wg_ref, b_ref,
             msg_ref, he_ref):
    u = u_ref[...]
    it = i_ref[...]
    ge = g_ref[...]
    m = jnp.dot(u, wu_ref[...], preferred_element_type=jnp.float32)
    m = m + jnp.dot(it, wi_ref[...], preferred_element_type=jnp.float32)
    m = m + jnp.dot(it * ge, wg_ref[...], preferred_element_type=jnp.float32)
    m = m + b_ref[...]
    msg_ref[...] = m
    he_ref[...] = hein_ref[...] + m


def _stage_b(umsg, imsg, gep, hein, W, b):
    wu, wi, wg = W[:D], W[D:2 * D], W[2 * D:]
    b2 = b.reshape(1, D)
    row_spec = pl.BlockSpec((RB, D), lambda ib: (ib, 0))
    w_spec = pl.BlockSpec((D, D), lambda ib: (0, 0))
    return pl.pallas_call(
        _mm_body,
        grid=(GP // RB,),
        in_specs=[row_spec, row_spec, row_spec, row_spec, w_spec, w_spec,
                  w_spec, pl.BlockSpec((1, D), lambda ib: (0, 0))],
        out_specs=[row_spec, row_spec],
        out_shape=(
            jax.ShapeDtypeStruct((GP, D), jnp.float32),
            jax.ShapeDtypeStruct((GP, D), jnp.float32),
        ),
    )(umsg, imsg, gep, hein, wu, wi, wg, b2)


def kernel(user_emb, item_emb, group_emb, uh_rows, uh_cols, uh_vals,
           ih_rows, ih_cols, ih_vals, fh_rows, fh_cols, fh_vals,
           W0, b0, W1, b1, num_users, num_items):
    i32 = jnp.int32

    def padded(x):
        return jnp.pad(x, (0, EPAD))

    uhr, uhc, uhv = padded(uh_rows), padded(uh_cols), padded(uh_vals)
    ihr, ihc, ihv = padded(ih_rows), padded(ih_cols), padded(ih_vals)
    fhr, fhc, fhv = padded(fh_rows), padded(fh_cols), padded(fh_vals)

    ub = jnp.pad(jnp.searchsorted(uh_rows, jnp.arange(NW + 1, dtype=i32) * WR_G)
                 .astype(i32), (0, 48 - (NW + 1)))
    ib = jnp.pad(jnp.searchsorted(ih_rows, jnp.arange(NW + 1, dtype=i32) * WR_G)
                 .astype(i32), (0, 48 - (NW + 1)))
    fb = jnp.pad(jnp.searchsorted(fh_rows, jnp.arange(NWIN_E + 1, dtype=i32) * WR_E)
                 .astype(i32), (0, 184 - (NWIN_E + 1)))

    base_p = jnp.pad(jnp.concatenate([user_emb, item_emb], axis=0),
                     ((0, EP - (U + I)), (0, 0)))
    gep = jnp.pad(group_emb, ((0, GP - G), (0, 0)))

    um1, im1 = _stage_a(0, uhr, uhc, uhv, ihr, ihc, ihv,
                        user_emb, item_emb, ub, ib)
    msg1, he1 = _stage_b(um1, im1, gep, gep, W0, b0)
    emb1, r1 = _stage_c(True, fhr, fhc, fhv, msg1, fb, base_p)

    um2, im2 = _stage_a(U, uhr, uhc, uhv, ihr, ihc, ihv, emb1, emb1, ub, ib)
    msg2, he2 = _stage_b(um2, im2, gep, he1, W1, b1)
    (final_p,) = _stage_c(False, fhr, fhc, fhv, msg2, fb, r1)

    return final_p[:U + I], he2[:G]
